# baseline (speedup 1.0000x reference)
import functools

import jax
import jax.numpy as jnp
from jax import lax
from jax.experimental import pallas as pl
from jax.experimental.pallas import tpu as pltpu

EPS = 1e-6
VN_NEG_SLOPE = 0.2
MLP_NEG_SLOPE = 0.01


def _cp(sem, vmem_mb=32):
    return pltpu.CompilerParams(dimension_semantics=sem,
                                vmem_limit_bytes=vmem_mb * 1024 * 1024)


def _pick_tile(m, cap=1024):
    top = min(m, cap)
    top -= top % 128
    for t in range(top, 0, -128):
        if m % t == 0:
            return t
    return m


def _nonlin(p, d, s, t):
    ssq = p[0] * p[0] + p[1] * p[1] + p[2] * p[2]
    norm = jnp.sqrt(ssq) + EPS
    fac = s + t * pl.reciprocal(norm, approx=True)
    p = [pa * fac for pa in p]
    dotpd = p[0] * d[0] + p[1] * d[1] + p[2] * d[2]
    dsq = d[0] * d[0] + d[1] * d[1] + d[2] * d[2]
    coef = dotpd * pl.reciprocal(dsq + EPS, approx=True)
    g = (1.0 - VN_NEG_SLOPE) * jnp.where(dotpd < 0.0, coef, 0.0)
    return [p[a] - g * d[a] for a in range(3)]


def _knn_kernel(xq_ref, x_ref, o_ref, *, n, k):
    bi = pl.program_id(0)
    xa = x_ref[...]
    g = None
    for a in range(3):
        ga = lax.dot_general(xq_ref[a], xa[a], (((0,), (0,)), ((), ())),
                             preferred_element_type=jnp.float32)
        g = ga if g is None else g + ga
    sq = jnp.sum(xa * xa, axis=(0, 1))[None, :]
    pd = 2.0 * g - sq
    flane = lax.broadcasted_iota(jnp.int32, pd.shape, 1).astype(jnp.float32)
    cols = []
    for _ in range(k):
        mk = jnp.max(pd, axis=1, keepdims=True)
        am = jnp.min(jnp.where(pd == mk, flane, jnp.inf), axis=1,
                     keepdims=True)
        cols.append(am.astype(jnp.int32))
        pd = jnp.where(flane == am, -jnp.inf, pd)
    o_ref[0] = jnp.concatenate(cols, axis=1) + bi * n


def _knn_gather(x_c, x_pm, k, b, n):
    c = x_c.shape[1]
    tq = _pick_tile(n, cap=256)
    nq = n // tq
    idx = pl.pallas_call(
        functools.partial(_knn_kernel, n=n, k=k),
        out_shape=jax.ShapeDtypeStruct((b, n, k), jnp.int32),
        grid=(b, nq),
        in_specs=[pl.BlockSpec((3, c, tq), lambda bi, q: (0, 0, bi * nq + q)),
                  pl.BlockSpec((3, c, n), lambda bi, q: (0, 0, bi))],
        out_specs=pl.BlockSpec((1, tq, k), lambda bi, q: (bi, q, 0)),
        compiler_params=_cp(("parallel", "parallel")),
    )(x_c, x_c)
    g = jnp.transpose(idx, (2, 0, 1)).reshape(-1)
    return jnp.take(x_pm, g, axis=0)


def _pm_rows(pm_t, c):
    return [pm_t[a * c:(a + 1) * c] for a in range(3)]


def _edge_mean_kernel(nbr_ref, ctr_ref, w_ref, s_ref, t_ref, mean_ref, pm_ref,
                      *, inv_k):
    kk = pl.program_id(1)
    w = w_ref[...]
    c = ctr_ref.shape[1]
    cout = w.shape[0] // 2
    nb = _pm_rows(jnp.transpose(nbr_ref[...]), c)
    p, d = [], []
    for a in range(3):
        cat = jnp.concatenate([nb[a], ctr_ref[a]], axis=0)
        pd = jnp.dot(w, cat, preferred_element_type=jnp.float32)
        p.append(pd[:cout])
        d.append(pd[cout:])
    o = _nonlin(p, d, s_ref[...], t_ref[...])

    @pl.when(kk == 0)
    def _():
        mean_ref[...] = jnp.zeros_like(mean_ref)

    for a in range(3):
        mean_ref[a] = mean_ref[a] + o[a]

    @pl.when(kk == pl.num_programs(1) - 1)
    def _():
        mean_ref[...] = mean_ref[...] * inv_k
        pm_ref[...] = jnp.concatenate(
            [jnp.transpose(mean_ref[a]) for a in range(3)], axis=1)


def _edge_mean(nbr_pm, ctr, w, s, t, k):
    _, c, m = ctr.shape
    cout = w.shape[0] // 2
    tm = _pick_tile(m)
    nt = m // tm
    return pl.pallas_call(
        functools.partial(_edge_mean_kernel, inv_k=1.0 / k),
        out_shape=(jax.ShapeDtypeStruct((3, cout, m), jnp.float32),
                   jax.ShapeDtypeStruct((m, 3 * cout), jnp.float32)),
        grid=(nt, k),
        in_specs=[pl.BlockSpec((tm, nbr_pm.shape[1]), lambda i, kk: (kk * nt + i, 0)),
                  pl.BlockSpec((3, c, tm), lambda i, kk: (0, 0, i)),
                  pl.BlockSpec(w.shape, lambda i, kk: (0, 0)),
                  pl.BlockSpec((cout, 1), lambda i, kk: (0, 0)),
                  pl.BlockSpec((cout, 1), lambda i, kk: (0, 0))],
        out_specs=[pl.BlockSpec((3, cout, tm), lambda i, kk: (0, 0, i)),
                   pl.BlockSpec((tm, 3 * cout), lambda i, kk: (i, 0))],
        compiler_params=_cp(("parallel", "arbitrary")),
    )(nbr_pm, ctr, w, s, t)


def _conv5_kernel(nbr1_ref, ctr1_ref, nbr2_ref, ctr2_ref, nbr3_ref, ctr3_ref,
                  nbr4_ref, ctr4_ref, w1_ref, wb_ref, w5_ref,
                  s1_ref, t1_ref, s2_ref, t2_ref, s3_ref, t3_ref, s4_ref, t4_ref,
                  s5_ref, t5_ref, mean_ref, *, inv_k, cout):
    kk = pl.program_id(1)
    w1 = w1_ref[...]
    wb = wb_ref[...]
    w5 = w5_ref[...]
    c = ctr2_ref.shape[1]
    st = [(s1_ref[...], t1_ref[...]), (s2_ref[...], t2_ref[...]),
          (s3_ref[...], t3_ref[...]), (s4_ref[...], t4_ref[...])]
    nb1 = _pm_rows(jnp.transpose(nbr1_ref[...]), 1)
    nb2 = _pm_rows(jnp.transpose(nbr2_ref[...]), c)
    nb3 = _pm_rows(jnp.transpose(nbr3_ref[...]), c)
    nb4 = _pm_rows(jnp.transpose(nbr4_ref[...]), c)
    pcols = [None] * 3
    dcols = [None] * 3
    for a in range(3):
        cat1 = jnp.concatenate([nb1[a], ctr1_ref[a]], axis=0)
        pd1 = jnp.dot(w1, cat1, preferred_element_type=jnp.float32)
        cat = jnp.concatenate([nb2[a], ctr2_ref[a], nb3[a], ctr3_ref[a],
                               nb4[a], ctr4_ref[a]], axis=0)
        pdb = jnp.dot(wb, cat, preferred_element_type=jnp.float32)
        pcols[a] = [pd1[:cout], pdb[:cout], pdb[2 * cout:3 * cout],
                    pdb[4 * cout:5 * cout]]
        dcols[a] = [pd1[cout:], pdb[cout:2 * cout], pdb[3 * cout:4 * cout],
                    pdb[5 * cout:]]
    o5 = []
    for i in range(4):
        s_i, t_i = st[i]
        o = _nonlin([pcols[a][i] for a in range(3)],
                    [dcols[a][i] for a in range(3)], s_i, t_i)
        o5.append(o)
    cout5 = w5.shape[0] // 2
    p5, d5 = [], []
    for a in range(3):
        cat5 = jnp.concatenate([o5[i][a] for i in range(4)], axis=0)
        pd = jnp.dot(w5, cat5, preferred_element_type=jnp.float32)
        p5.append(pd[:cout5])
        d5.append(pd[cout5:])
    o = _nonlin(p5, d5, s5_ref[...], t5_ref[...])

    @pl.when(kk == 0)
    def _():
        mean_ref[...] = jnp.zeros_like(mean_ref)

    for a in range(3):
        mean_ref[a] = mean_ref[a] + o[a]

    @pl.when(kk == pl.num_programs(1) - 1)
    def _():
        mean_ref[...] = mean_ref[...] * inv_k


def _conv5(nbrs, ctrs, w1, wb, w5, sts, s5, t5, k):
    _, c, m = ctrs[1].shape
    cout5 = w5.shape[0] // 2
    cout = wb.shape[0] // 6
    tm = _pick_tile(m)
    nt = m // tm
    nbr_spec = lambda arr: pl.BlockSpec((tm, arr.shape[1]), lambda i, kk: (kk * nt + i, 0))
    ctr_spec = lambda cc: pl.BlockSpec((3, cc, tm), lambda i, kk: (0, 0, i))
    wspec = lambda shp: pl.BlockSpec(shp, lambda i, kk: tuple(0 for _ in shp))
    in_specs = [nbr_spec(nbrs[0]), ctr_spec(1), nbr_spec(nbrs[1]), ctr_spec(c),
                nbr_spec(nbrs[2]), ctr_spec(c), nbr_spec(nbrs[3]), ctr_spec(c),
                wspec(w1.shape), wspec(wb.shape), wspec(w5.shape)]
    st_args = []
    for s_i, t_i in sts:
        in_specs += [wspec(s_i.shape), wspec(t_i.shape)]
        st_args += [s_i, t_i]
    in_specs += [wspec(s5.shape), wspec(t5.shape)]
    return pl.pallas_call(
        functools.partial(_conv5_kernel, inv_k=1.0 / k, cout=cout),
        out_shape=jax.ShapeDtypeStruct((3, cout5, m), jnp.float32),
        grid=(nt, k),
        in_specs=in_specs,
        out_specs=pl.BlockSpec((3, cout5, tm), lambda i, kk: (0, 0, i)),
        compiler_params=_cp(("parallel", "arbitrary")),
    )(nbrs[0], ctrs[0], nbrs[1], ctrs[1], nbrs[2], ctrs[2], nbrs[3], ctrs[3],
      w1, wb, w5, *st_args, s5, t5)


def _tail_kernel(x5_ref, w6_ref, s6_ref, t6_ref, wv1_ref, sv1_ref, tv1_ref,
                 wv2_ref, sv2_ref, tv2_ref, wlin_ref, w1a_ref, w1b_ref, b1_ref,
                 s1_ref, t1_ref, w2_ref, b2_ref, s2_ref, t2_ref, w3_ref, b3_ref,
                 o_ref, *, inv_n, c6):
    w6 = w6_ref[...]
    p6, d6 = [], []
    for a in range(3):
        pd = jnp.dot(w6, x5_ref[a], preferred_element_type=jnp.float32)
        p6.append(pd[:c6])
        d6.append(pd[c6:])
    x6 = _nonlin(p6, d6, s6_ref[...], t6_ref[...])
    xc = []
    for a in range(3):
        xm = jnp.sum(x6[a], axis=-1, keepdims=True) * inv_n
        xc.append(jnp.concatenate(
            [x6[a], jnp.broadcast_to(xm, x6[a].shape)], axis=0))

    wv1 = wv1_ref[...]
    cv1 = wv1.shape[0] // 2
    p, d = [], []
    for a in range(3):
        pd = jnp.dot(wv1, xc[a], preferred_element_type=jnp.float32)
        p.append(pd[:cv1])
        d.append(pd[cv1:])
    z = _nonlin(p, d, sv1_ref[...], tv1_ref[...])

    wv2 = wv2_ref[...]
    cv2 = wv2.shape[0] // 2
    p, d = [], []
    for a in range(3):
        pd = jnp.dot(wv2, z[a], preferred_element_type=jnp.float32)
        p.append(pd[:cv2])
        d.append(pd[cv2:])
    z = _nonlin(p, d, sv2_ref[...], tv2_ref[...])

    w_lin = wlin_ref[...]
    r = [jnp.dot(w_lin, z[a], preferred_element_type=jnp.float32) for a in range(3)]
    cdims = (((0,), (0,)), ((), ()))
    h = b1_ref[...]
    for kk in range(3):
        xstd = (xc[0] * r[0][kk:kk + 1, :] + xc[1] * r[1][kk:kk + 1, :]
                + xc[2] * r[2][kk:kk + 1, :])
        mx = jnp.max(xstd, axis=-1, keepdims=True)
        av = jnp.sum(xstd, axis=-1, keepdims=True) * inv_n
        h = h + lax.dot_general(mx, w1a_ref[kk], cdims,
                                preferred_element_type=jnp.float32)
        h = h + lax.dot_general(av, w1b_ref[kk], cdims,
                                preferred_element_type=jnp.float32)
    h = h * s1_ref[...] + t1_ref[...]
    h = jnp.where(h >= 0.0, h, MLP_NEG_SLOPE * h)
    h = jnp.dot(h, w2_ref[...], preferred_element_type=jnp.float32) + b2_ref[...]
    h = h * s2_ref[...] + t2_ref[...]
    h = jnp.where(h >= 0.0, h, MLP_NEG_SLOPE * h)
    o_ref[0] = jnp.dot(h, w3_ref[...], preferred_element_type=jnp.float32) + b3_ref[...]


def _tail(x5, w6, s6, t6, wv1, sv1, tv1, wv2, sv2, tv2, hp, b, n):
    _, c5, m = x5.shape
    c6 = w6.shape[0] - 1
    nc = hp['w3'].shape[1]
    ws = lambda shp: pl.BlockSpec(shp, lambda bi: tuple(0 for _ in shp))
    out = pl.pallas_call(
        functools.partial(_tail_kernel, inv_n=1.0 / n, c6=c6),
        out_shape=jax.ShapeDtypeStruct((b, 1, nc), jnp.float32),
        grid=(b,),
        in_specs=[pl.BlockSpec((3, c5, n), lambda bi: (0, 0, bi)),
                  ws(w6.shape), ws(s6.shape), ws(t6.shape),
                  ws(wv1.shape), ws(sv1.shape), ws(tv1.shape),
                  ws(wv2.shape), ws(sv2.shape), ws(tv2.shape),
                  ws(hp['std_lin'].shape), ws(hp['w1a'].shape), ws(hp['w1b'].shape),
                  ws(hp['b1'].shape), ws(hp['s1'].shape), ws(hp['t1'].shape),
                  ws(hp['w2'].shape), ws(hp['b2'].shape), ws(hp['s2'].shape),
                  ws(hp['t2'].shape), ws(hp['w3'].shape), ws(hp['b3'].shape)],
        out_specs=pl.BlockSpec((1, 1, nc), lambda bi: (bi, 0, 0)),
        compiler_params=_cp(("parallel",)),
    )(x5, w6, s6, t6, wv1, sv1, tv1, wv2, sv2, tv2,
      hp['std_lin'], hp['w1a'], hp['w1b'], hp['b1'], hp['s1'], hp['t1'],
      hp['w2'], hp['b2'], hp['s2'], hp['t2'], hp['w3'], hp['b3'])
    return out[:, 0, :]


def _stack_edge_w(wf_n, wf_c, wd_n, wd_c):
    return jnp.concatenate([jnp.concatenate([wf_n, wf_c], axis=1),
                            jnp.concatenate([wd_n, wd_c], axis=1)], axis=0)


def kernel(points,
           conv1_wf_n, conv1_wf_c, conv1_wd_n, conv1_wd_c, conv1_s, conv1_t,
           conv2_wf_n, conv2_wf_c, conv2_wd_n, conv2_wd_c, conv2_s, conv2_t,
           conv3_wf_n, conv3_wf_c, conv3_wd_n, conv3_wd_c, conv3_s, conv3_t,
           conv4_wf_n, conv4_wf_c, conv4_wd_n, conv4_wd_c, conv4_s, conv4_t,
           conv5_wf, conv5_wd, conv5_s, conv5_t,
           conv6_wf, conv6_wd, conv6_s, conv6_t,
           stdvn1_wf, stdvn1_wd, stdvn1_s, stdvn1_t,
           stdvn2_wf, stdvn2_wd, stdvn2_s, stdvn2_t,
           head_std_lin, head_w1a, head_w1b, head_b1, head_s1, head_t1,
           head_w2, head_b2, head_s2, head_t2, head_w3, head_b3):
    k = 8
    b, three, n = points.shape
    m = b * n
    x = jnp.transpose(points, (1, 0, 2)).reshape(3, 1, m)
    x_pm = jnp.pad(jnp.transpose(points, (0, 2, 1)).reshape(m, 3),
                   ((0, 0), (0, 5)))

    w1 = _stack_edge_w(conv1_wf_n, conv1_wf_c, conv1_wd_n, conv1_wd_c)
    w2 = _stack_edge_w(conv2_wf_n, conv2_wf_c, conv2_wd_n, conv2_wd_c)
    w3 = _stack_edge_w(conv3_wf_n, conv3_wf_c, conv3_wd_n, conv3_wd_c)
    w4 = _stack_edge_w(conv4_wf_n, conv4_wf_c, conv4_wd_n, conv4_wd_c)
    cout = conv2_wf_n.shape[0]
    cc = conv2_wf_n.shape[1]
    wb = jnp.zeros((6 * cout, 6 * cc), jnp.float32)
    wb = wb.at[0:2 * cout, 0:2 * cc].set(w2)
    wb = wb.at[2 * cout:4 * cout, 2 * cc:4 * cc].set(w3)
    wb = wb.at[4 * cout:6 * cout, 4 * cc:6 * cc].set(w4)
    cout5 = conv5_wf.shape[1]
    w5 = jnp.concatenate(
        [jnp.transpose(conv5_wf, (1, 0, 2)).reshape(cout5, -1),
         jnp.transpose(conv5_wd, (1, 0, 2)).reshape(cout5, -1)], axis=0)

    nbr1 = _knn_gather(x, x_pm, k, b, n)
    x1, x1_pm = _edge_mean(nbr1, x, w1, conv1_s, conv1_t, k)
    nbr2 = _knn_gather(x1, x1_pm, k, b, n)
    x2, x2_pm = _edge_mean(nbr2, x1, w2, conv2_s, conv2_t, k)
    nbr3 = _knn_gather(x2, x2_pm, k, b, n)
    x3, x3_pm = _edge_mean(nbr3, x2, w3, conv3_s, conv3_t, k)
    nbr4 = _knn_gather(x3, x3_pm, k, b, n)

    x5 = _conv5([nbr1, nbr2, nbr3, nbr4], [x, x1, x2, x3], w1, wb, w5,
                [(conv1_s, conv1_t), (conv2_s, conv2_t),
                 (conv3_s, conv3_t), (conv4_s, conv4_t)],
                conv5_s, conv5_t, k)

    w6 = jnp.concatenate([conv6_wf, conv6_wd], axis=0)
    wv1 = jnp.concatenate([stdvn1_wf, stdvn1_wd], axis=0)
    wv2 = jnp.concatenate([stdvn2_wf, stdvn2_wd], axis=0)
    hp = dict(std_lin=head_std_lin, w1a=head_w1a, w1b=head_w1b, b1=head_b1,
              s1=head_s1, t1=head_t1, w2=head_w2, b2=head_b2, s2=head_s2,
              t2=head_t2, w3=head_w3, b3=head_b3)
    return _tail(x5, w6, conv6_s, conv6_t, wv1, stdvn1_s, stdvn1_t,
                 wv2, stdvn2_s, stdvn2_t, hp, b, n)


# speedup vs baseline: 1.0108x; 1.0108x over previous
import functools

import jax
import jax.numpy as jnp
from jax import lax
from jax.experimental import pallas as pl
from jax.experimental.pallas import tpu as pltpu

EPS = 1e-6
VN_NEG_SLOPE = 0.2
MLP_NEG_SLOPE = 0.01


def _cp(sem, vmem_mb=32):
    return pltpu.CompilerParams(dimension_semantics=sem,
                                vmem_limit_bytes=vmem_mb * 1024 * 1024)


def _pick_tile(m, cap=1024):
    top = min(m, cap)
    top -= top % 128
    for t in range(top, 0, -128):
        if m % t == 0:
            return t
    return m


def _nonlin(p, d, s, t):
    ssq = p[0] * p[0] + p[1] * p[1] + p[2] * p[2]
    norm = jnp.sqrt(ssq) + EPS
    fac = s + t * pl.reciprocal(norm, approx=True)
    p = [pa * fac for pa in p]
    dotpd = p[0] * d[0] + p[1] * d[1] + p[2] * d[2]
    dsq = d[0] * d[0] + d[1] * d[1] + d[2] * d[2]
    coef = dotpd * pl.reciprocal(dsq + EPS, approx=True)
    g = (1.0 - VN_NEG_SLOPE) * jnp.where(dotpd < 0.0, coef, 0.0)
    return [p[a] - g * d[a] for a in range(3)]


def _knn_kernel(xq_ref, x_ref, o_ref, *, n, k):
    bi = pl.program_id(0)
    xa = x_ref[...]
    g = None
    for a in range(3):
        ga = lax.dot_general(xq_ref[a], xa[a], (((0,), (0,)), ((), ())),
                             preferred_element_type=jnp.float32)
        g = ga if g is None else g + ga
    sq = jnp.sum(xa * xa, axis=(0, 1))[None, :]
    pd = 2.0 * g - sq
    flane = lax.broadcasted_iota(jnp.int32, pd.shape, 1).astype(jnp.float32)
    cols = []
    for _ in range(k):
        mk = jnp.max(pd, axis=1, keepdims=True)
        am = jnp.min(jnp.where(pd == mk, flane, jnp.inf), axis=1,
                     keepdims=True)
        cols.append(am.astype(jnp.int32))
        pd = jnp.where(flane == am, -jnp.inf, pd)
    o_ref[0] = jnp.concatenate(cols, axis=1) + bi * n


def _knn_gather(x_c, x_pm, k, b, n):
    c = x_c.shape[1]
    tq = _pick_tile(n, cap=256)
    nq = n // tq
    idx = pl.pallas_call(
        functools.partial(_knn_kernel, n=n, k=k),
        out_shape=jax.ShapeDtypeStruct((b, n, k), jnp.int32),
        grid=(b, nq),
        in_specs=[pl.BlockSpec((3, c, tq), lambda bi, q: (0, 0, bi * nq + q)),
                  pl.BlockSpec((3, c, n), lambda bi, q: (0, 0, bi))],
        out_specs=pl.BlockSpec((1, tq, k), lambda bi, q: (bi, q, 0)),
        compiler_params=_cp(("parallel", "parallel")),
    )(x_c, x_c)
    g = jnp.transpose(idx, (2, 0, 1)).reshape(-1)
    return jnp.take(x_pm, g, axis=0)


def _pm_rows(pm_t, c):
    return [pm_t[a * c:(a + 1) * c] for a in range(3)]


def _edge_mean_kernel(nbr_ref, ctr_ref, w_ref, s_ref, t_ref, mean_ref, pm_ref,
                      *, inv_k):
    kk = pl.program_id(1)
    w = w_ref[...]
    c = ctr_ref.shape[1]
    cout = w.shape[0] // 2
    nb = _pm_rows(jnp.transpose(nbr_ref[...]), c)
    p, d = [], []
    for a in range(3):
        cat = jnp.concatenate([nb[a], ctr_ref[a]], axis=0)
        pd = jnp.dot(w, cat, preferred_element_type=jnp.float32)
        p.append(pd[:cout])
        d.append(pd[cout:])
    o = _nonlin(p, d, s_ref[...], t_ref[...])

    @pl.when(kk == 0)
    def _():
        mean_ref[...] = jnp.zeros_like(mean_ref)

    for a in range(3):
        mean_ref[a] = mean_ref[a] + o[a]

    @pl.when(kk == pl.num_programs(1) - 1)
    def _():
        mean_ref[...] = mean_ref[...] * inv_k
        pm_ref[...] = jnp.concatenate(
            [jnp.transpose(mean_ref[a]) for a in range(3)], axis=1)


def _edge_mean(nbr_pm, ctr, w, s, t, k):
    _, c, m = ctr.shape
    cout = w.shape[0] // 2
    tm = _pick_tile(m)
    nt = m // tm
    return pl.pallas_call(
        functools.partial(_edge_mean_kernel, inv_k=1.0 / k),
        out_shape=(jax.ShapeDtypeStruct((3, cout, m), jnp.float32),
                   jax.ShapeDtypeStruct((m, 3 * cout), jnp.float32)),
        grid=(nt, k),
        in_specs=[pl.BlockSpec((tm, nbr_pm.shape[1]), lambda i, kk: (kk * nt + i, 0)),
                  pl.BlockSpec((3, c, tm), lambda i, kk: (0, 0, i)),
                  pl.BlockSpec(w.shape, lambda i, kk: (0, 0)),
                  pl.BlockSpec((cout, 1), lambda i, kk: (0, 0)),
                  pl.BlockSpec((cout, 1), lambda i, kk: (0, 0))],
        out_specs=[pl.BlockSpec((3, cout, tm), lambda i, kk: (0, 0, i)),
                   pl.BlockSpec((tm, 3 * cout), lambda i, kk: (i, 0))],
        compiler_params=_cp(("parallel", "arbitrary")),
    )(nbr_pm, ctr, w, s, t)


def _conv5_kernel(nbr1_ref, ctr1_ref, nbr2_ref, ctr2_ref, nbr3_ref, ctr3_ref,
                  nbr4_ref, ctr4_ref, w1_ref, wb_ref, w5_ref,
                  s1_ref, t1_ref, s2_ref, t2_ref, s3_ref, t3_ref, s4_ref, t4_ref,
                  s5_ref, t5_ref, mean_ref, *, inv_k, cout):
    kk = pl.program_id(1)
    w1 = w1_ref[...]
    wb = wb_ref[...]
    w5 = w5_ref[...]
    c = ctr2_ref.shape[1]
    st = [(s1_ref[...], t1_ref[...]), (s2_ref[...], t2_ref[...]),
          (s3_ref[...], t3_ref[...]), (s4_ref[...], t4_ref[...])]
    nb1 = _pm_rows(jnp.transpose(nbr1_ref[...]), 1)
    nb2 = _pm_rows(jnp.transpose(nbr2_ref[...]), c)
    nb3 = _pm_rows(jnp.transpose(nbr3_ref[...]), c)
    nb4 = _pm_rows(jnp.transpose(nbr4_ref[...]), c)
    pcols = [None] * 3
    dcols = [None] * 3
    for a in range(3):
        cat1 = jnp.concatenate([nb1[a], ctr1_ref[a]], axis=0)
        pd1 = jnp.dot(w1, cat1, preferred_element_type=jnp.float32)
        cat = jnp.concatenate([nb2[a], ctr2_ref[a], nb3[a], ctr3_ref[a],
                               nb4[a], ctr4_ref[a]], axis=0)
        pdb = jnp.dot(wb, cat, preferred_element_type=jnp.float32)
        pcols[a] = [pd1[:cout], pdb[:cout], pdb[2 * cout:3 * cout],
                    pdb[4 * cout:5 * cout]]
        dcols[a] = [pd1[cout:], pdb[cout:2 * cout], pdb[3 * cout:4 * cout],
                    pdb[5 * cout:]]
    o5 = []
    for i in range(4):
        s_i, t_i = st[i]
        o = _nonlin([pcols[a][i] for a in range(3)],
                    [dcols[a][i] for a in range(3)], s_i, t_i)
        o5.append(o)
    cout5 = w5.shape[0] // 2
    p5, d5 = [], []
    for a in range(3):
        cat5 = jnp.concatenate([o5[i][a] for i in range(4)], axis=0)
        pd = jnp.dot(w5, cat5, preferred_element_type=jnp.float32)
        p5.append(pd[:cout5])
        d5.append(pd[cout5:])
    o = _nonlin(p5, d5, s5_ref[...], t5_ref[...])

    @pl.when(kk == 0)
    def _():
        mean_ref[...] = jnp.zeros_like(mean_ref)

    for a in range(3):
        mean_ref[a] = mean_ref[a] + o[a]

    @pl.when(kk == pl.num_programs(1) - 1)
    def _():
        mean_ref[...] = mean_ref[...] * inv_k


def _conv5(nbrs, ctrs, w1, wb, w5, sts, s5, t5, k):
    _, c, m = ctrs[1].shape
    cout5 = w5.shape[0] // 2
    cout = wb.shape[0] // 6
    tm = _pick_tile(m)
    nt = m // tm
    nbr_spec = lambda arr: pl.BlockSpec((tm, arr.shape[1]), lambda i, kk: (kk * nt + i, 0))
    ctr_spec = lambda cc: pl.BlockSpec((3, cc, tm), lambda i, kk: (0, 0, i))
    wspec = lambda shp: pl.BlockSpec(shp, lambda i, kk: tuple(0 for _ in shp))
    in_specs = [nbr_spec(nbrs[0]), ctr_spec(1), nbr_spec(nbrs[1]), ctr_spec(c),
                nbr_spec(nbrs[2]), ctr_spec(c), nbr_spec(nbrs[3]), ctr_spec(c),
                wspec(w1.shape), wspec(wb.shape), wspec(w5.shape)]
    st_args = []
    for s_i, t_i in sts:
        in_specs += [wspec(s_i.shape), wspec(t_i.shape)]
        st_args += [s_i, t_i]
    in_specs += [wspec(s5.shape), wspec(t5.shape)]
    return pl.pallas_call(
        functools.partial(_conv5_kernel, inv_k=1.0 / k, cout=cout),
        out_shape=jax.ShapeDtypeStruct((3, cout5, m), jnp.float32),
        grid=(nt, k),
        in_specs=in_specs,
        out_specs=pl.BlockSpec((3, cout5, tm), lambda i, kk: (0, 0, i)),
        compiler_params=_cp(("parallel", "arbitrary")),
    )(nbrs[0], ctrs[0], nbrs[1], ctrs[1], nbrs[2], ctrs[2], nbrs[3], ctrs[3],
      w1, wb, w5, *st_args, s5, t5)


def _tail_kernel(x5_ref, w6_ref, s6_ref, t6_ref, wv1_ref, sv1_ref, tv1_ref,
                 wv2_ref, sv2_ref, tv2_ref, wlin_ref, w1a_ref, w1b_ref, b1_ref,
                 s1_ref, t1_ref, w2_ref, b2_ref, s2_ref, t2_ref, w3_ref, b3_ref,
                 o_ref, *, inv_n, c6):
    w6 = w6_ref[...]
    p6, d6 = [], []
    for a in range(3):
        pd = jnp.dot(w6, x5_ref[a], preferred_element_type=jnp.float32)
        p6.append(pd[:c6])
        d6.append(pd[c6:])
    x6 = _nonlin(p6, d6, s6_ref[...], t6_ref[...])
    xc = []
    for a in range(3):
        xm = jnp.sum(x6[a], axis=-1, keepdims=True) * inv_n
        xc.append(jnp.concatenate(
            [x6[a], jnp.broadcast_to(xm, x6[a].shape)], axis=0))

    wv1 = wv1_ref[...]
    cv1 = wv1.shape[0] // 2
    p, d = [], []
    for a in range(3):
        pd = jnp.dot(wv1, xc[a], preferred_element_type=jnp.float32)
        p.append(pd[:cv1])
        d.append(pd[cv1:])
    z = _nonlin(p, d, sv1_ref[...], tv1_ref[...])

    wv2 = wv2_ref[...]
    cv2 = wv2.shape[0] // 2
    p, d = [], []
    for a in range(3):
        pd = jnp.dot(wv2, z[a], preferred_element_type=jnp.float32)
        p.append(pd[:cv2])
        d.append(pd[cv2:])
    z = _nonlin(p, d, sv2_ref[...], tv2_ref[...])

    w_lin = wlin_ref[...]
    r = [jnp.dot(w_lin, z[a], preferred_element_type=jnp.float32) for a in range(3)]
    cdims = (((0,), (0,)), ((), ()))
    h = b1_ref[...]
    for kk in range(3):
        xstd = (xc[0] * r[0][kk:kk + 1, :] + xc[1] * r[1][kk:kk + 1, :]
                + xc[2] * r[2][kk:kk + 1, :])
        mx = jnp.max(xstd, axis=-1, keepdims=True)
        av = jnp.sum(xstd, axis=-1, keepdims=True) * inv_n
        h = h + lax.dot_general(mx, w1a_ref[kk], cdims,
                                preferred_element_type=jnp.float32)
        h = h + lax.dot_general(av, w1b_ref[kk], cdims,
                                preferred_element_type=jnp.float32)
    h = h * s1_ref[...] + t1_ref[...]
    h = jnp.where(h >= 0.0, h, MLP_NEG_SLOPE * h)
    h = jnp.dot(h, w2_ref[...], preferred_element_type=jnp.float32) + b2_ref[...]
    h = h * s2_ref[...] + t2_ref[...]
    h = jnp.where(h >= 0.0, h, MLP_NEG_SLOPE * h)
    o_ref[0] = jnp.dot(h, w3_ref[...], preferred_element_type=jnp.float32) + b3_ref[...]


def _tail(x5, w6, s6, t6, wv1, sv1, tv1, wv2, sv2, tv2, hp, b, n):
    _, c5, m = x5.shape
    c6 = w6.shape[0] - 1
    nc = hp['w3'].shape[1]
    ws = lambda shp: pl.BlockSpec(shp, lambda bi: tuple(0 for _ in shp))
    out = pl.pallas_call(
        functools.partial(_tail_kernel, inv_n=1.0 / n, c6=c6),
        out_shape=jax.ShapeDtypeStruct((b, 1, nc), jnp.float32),
        grid=(b,),
        in_specs=[pl.BlockSpec((3, c5, n), lambda bi: (0, 0, bi)),
                  ws(w6.shape), ws(s6.shape), ws(t6.shape),
                  ws(wv1.shape), ws(sv1.shape), ws(tv1.shape),
                  ws(wv2.shape), ws(sv2.shape), ws(tv2.shape),
                  ws(hp['std_lin'].shape), ws(hp['w1a'].shape), ws(hp['w1b'].shape),
                  ws(hp['b1'].shape), ws(hp['s1'].shape), ws(hp['t1'].shape),
                  ws(hp['w2'].shape), ws(hp['b2'].shape), ws(hp['s2'].shape),
                  ws(hp['t2'].shape), ws(hp['w3'].shape), ws(hp['b3'].shape)],
        out_specs=pl.BlockSpec((1, 1, nc), lambda bi: (bi, 0, 0)),
        compiler_params=_cp(("parallel",)),
    )(x5, w6, s6, t6, wv1, sv1, tv1, wv2, sv2, tv2,
      hp['std_lin'], hp['w1a'], hp['w1b'], hp['b1'], hp['s1'], hp['t1'],
      hp['w2'], hp['b2'], hp['s2'], hp['t2'], hp['w3'], hp['b3'])
    return out[:, 0, :]


def _stack_edge_w(wf_n, wf_c, wd_n, wd_c):
    return jnp.concatenate([jnp.concatenate([wf_n, wf_c], axis=1),
                            jnp.concatenate([wd_n, wd_c], axis=1)], axis=0)


def kernel(points,
           conv1_wf_n, conv1_wf_c, conv1_wd_n, conv1_wd_c, conv1_s, conv1_t,
           conv2_wf_n, conv2_wf_c, conv2_wd_n, conv2_wd_c, conv2_s, conv2_t,
           conv3_wf_n, conv3_wf_c, conv3_wd_n, conv3_wd_c, conv3_s, conv3_t,
           conv4_wf_n, conv4_wf_c, conv4_wd_n, conv4_wd_c, conv4_s, conv4_t,
           conv5_wf, conv5_wd, conv5_s, conv5_t,
           conv6_wf, conv6_wd, conv6_s, conv6_t,
           stdvn1_wf, stdvn1_wd, stdvn1_s, stdvn1_t,
           stdvn2_wf, stdvn2_wd, stdvn2_s, stdvn2_t,
           head_std_lin, head_w1a, head_w1b, head_b1, head_s1, head_t1,
           head_w2, head_b2, head_s2, head_t2, head_w3, head_b3):
    k = 8
    b, three, n = points.shape
    m = b * n
    x = jnp.transpose(points, (1, 0, 2)).reshape(3, 1, m)
    x_pm = lax.optimization_barrier(
        jnp.pad(jnp.transpose(points, (0, 2, 1)).reshape(m, 3),
                ((0, 0), (0, 5))))

    w1 = _stack_edge_w(conv1_wf_n, conv1_wf_c, conv1_wd_n, conv1_wd_c)
    w2 = _stack_edge_w(conv2_wf_n, conv2_wf_c, conv2_wd_n, conv2_wd_c)
    w3 = _stack_edge_w(conv3_wf_n, conv3_wf_c, conv3_wd_n, conv3_wd_c)
    w4 = _stack_edge_w(conv4_wf_n, conv4_wf_c, conv4_wd_n, conv4_wd_c)
    cout = conv2_wf_n.shape[0]
    cc = conv2_wf_n.shape[1]
    wb = jnp.zeros((6 * cout, 6 * cc), jnp.float32)
    wb = wb.at[0:2 * cout, 0:2 * cc].set(w2)
    wb = wb.at[2 * cout:4 * cout, 2 * cc:4 * cc].set(w3)
    wb = wb.at[4 * cout:6 * cout, 4 * cc:6 * cc].set(w4)
    cout5 = conv5_wf.shape[1]
    w5 = jnp.concatenate(
        [jnp.transpose(conv5_wf, (1, 0, 2)).reshape(cout5, -1),
         jnp.transpose(conv5_wd, (1, 0, 2)).reshape(cout5, -1)], axis=0)

    nbr1 = _knn_gather(x, x_pm, k, b, n)
    x1, x1_pm = _edge_mean(nbr1, x, w1, conv1_s, conv1_t, k)
    nbr2 = _knn_gather(x1, x1_pm, k, b, n)
    x2, x2_pm = _edge_mean(nbr2, x1, w2, conv2_s, conv2_t, k)
    nbr3 = _knn_gather(x2, x2_pm, k, b, n)
    x3, x3_pm = _edge_mean(nbr3, x2, w3, conv3_s, conv3_t, k)
    nbr4 = _knn_gather(x3, x3_pm, k, b, n)

    x5 = _conv5([nbr1, nbr2, nbr3, nbr4], [x, x1, x2, x3], w1, wb, w5,
                [(conv1_s, conv1_t), (conv2_s, conv2_t),
                 (conv3_s, conv3_t), (conv4_s, conv4_t)],
                conv5_s, conv5_t, k)

    w6 = jnp.concatenate([conv6_wf, conv6_wd], axis=0)
    wv1 = jnp.concatenate([stdvn1_wf, stdvn1_wd], axis=0)
    wv2 = jnp.concatenate([stdvn2_wf, stdvn2_wd], axis=0)
    hp = dict(std_lin=head_std_lin, w1a=head_w1a, w1b=head_w1b, b1=head_b1,
              s1=head_s1, t1=head_t1, w2=head_w2, b2=head_b2, s2=head_s2,
              t2=head_t2, w3=head_w3, b3=head_b3)
    return _tail(x5, w6, conv6_s, conv6_t, wv1, stdvn1_s, stdvn1_t,
                 wv2, stdvn2_s, stdvn2_t, hp, b, n)


# speedup vs baseline: 5.0383x; 4.9844x over previous
import functools

import jax
import jax.numpy as jnp
from jax import lax
from jax.experimental import pallas as pl
from jax.experimental.pallas import tpu as pltpu

EPS = 1e-6
VN_NEG_SLOPE = 0.2
MLP_NEG_SLOPE = 0.01


def _cp(sem, vmem_mb=32):
    return pltpu.CompilerParams(dimension_semantics=sem,
                                vmem_limit_bytes=vmem_mb * 1024 * 1024)


def _pick_tile(m, cap=1024):
    top = min(m, cap)
    top -= top % 128
    for t in range(top, 0, -128):
        if m % t == 0:
            return t
    return m


def _nonlin(p, d, s, t):
    ssq = p[0] * p[0] + p[1] * p[1] + p[2] * p[2]
    norm = jnp.sqrt(ssq) + EPS
    fac = s + t * pl.reciprocal(norm, approx=True)
    p = [pa * fac for pa in p]
    dotpd = p[0] * d[0] + p[1] * d[1] + p[2] * d[2]
    dsq = d[0] * d[0] + d[1] * d[1] + d[2] * d[2]
    coef = dotpd * pl.reciprocal(dsq + EPS, approx=True)
    g = (1.0 - VN_NEG_SLOPE) * jnp.where(dotpd < 0.0, coef, 0.0)
    return [p[a] - g * d[a] for a in range(3)]


def _knn_kernel(xq_ref, x_ref, xpm_ref, o_ref, *, n, k):
    xa = x_ref[...]
    xpm = xpm_ref[...]
    g = None
    for a in range(3):
        ga = lax.dot_general(xq_ref[a], xa[a], (((0,), (0,)), ((), ())),
                             preferred_element_type=jnp.float32)
        g = ga if g is None else g + ga
    sq = jnp.sum(xa * xa, axis=(0, 1))[None, :]
    pd = 2.0 * g - sq
    flane = lax.broadcasted_iota(jnp.int32, pd.shape, 1).astype(jnp.float32)
    for j in range(k):
        mk = jnp.max(pd, axis=1, keepdims=True)
        am = jnp.min(jnp.where(pd == mk, flane, jnp.inf), axis=1,
                     keepdims=True)
        msk = flane == am
        o_ref[j] = jnp.dot(msk.astype(jnp.float32), xpm,
                           preferred_element_type=jnp.float32)
        pd = jnp.where(msk, -jnp.inf, pd)


def _knn_gather(x_c, x_pm, k, b, n):
    c = x_c.shape[1]
    w = x_pm.shape[1]
    m = b * n
    tq = _pick_tile(n, cap=256)
    nq = n // tq
    vals = pl.pallas_call(
        functools.partial(_knn_kernel, n=n, k=k),
        out_shape=jax.ShapeDtypeStruct((k, m, w), jnp.float32),
        grid=(b, nq),
        in_specs=[pl.BlockSpec((3, c, tq), lambda bi, q: (0, 0, bi * nq + q)),
                  pl.BlockSpec((3, c, n), lambda bi, q: (0, 0, bi)),
                  pl.BlockSpec((n, w), lambda bi, q: (bi, 0))],
        out_specs=pl.BlockSpec((k, tq, w), lambda bi, q: (0, bi * nq + q, 0)),
        compiler_params=_cp(("parallel", "parallel")),
    )(x_c, x_c, x_pm)
    return vals.reshape(k * m, w)


def _pm_rows(pm_t, c):
    return [pm_t[a * c:(a + 1) * c] for a in range(3)]


def _edge_mean_kernel(nbr_ref, ctr_ref, w_ref, s_ref, t_ref, mean_ref, pm_ref,
                      *, inv_k):
    kk = pl.program_id(1)
    w = w_ref[...]
    c = ctr_ref.shape[1]
    cout = w.shape[0] // 2
    nb = _pm_rows(jnp.transpose(nbr_ref[...]), c)
    p, d = [], []
    for a in range(3):
        cat = jnp.concatenate([nb[a], ctr_ref[a]], axis=0)
        pd = jnp.dot(w, cat, preferred_element_type=jnp.float32)
        p.append(pd[:cout])
        d.append(pd[cout:])
    o = _nonlin(p, d, s_ref[...], t_ref[...])

    @pl.when(kk == 0)
    def _():
        mean_ref[...] = jnp.zeros_like(mean_ref)

    for a in range(3):
        mean_ref[a] = mean_ref[a] + o[a]

    @pl.when(kk == pl.num_programs(1) - 1)
    def _():
        mean_ref[...] = mean_ref[...] * inv_k
        pm_ref[...] = jnp.concatenate(
            [jnp.transpose(mean_ref[a]) for a in range(3)], axis=1)


def _edge_mean(nbr_pm, ctr, w, s, t, k):
    _, c, m = ctr.shape
    cout = w.shape[0] // 2
    tm = _pick_tile(m)
    nt = m // tm
    return pl.pallas_call(
        functools.partial(_edge_mean_kernel, inv_k=1.0 / k),
        out_shape=(jax.ShapeDtypeStruct((3, cout, m), jnp.float32),
                   jax.ShapeDtypeStruct((m, 3 * cout), jnp.float32)),
        grid=(nt, k),
        in_specs=[pl.BlockSpec((tm, nbr_pm.shape[1]), lambda i, kk: (kk * nt + i, 0)),
                  pl.BlockSpec((3, c, tm), lambda i, kk: (0, 0, i)),
                  pl.BlockSpec(w.shape, lambda i, kk: (0, 0)),
                  pl.BlockSpec((cout, 1), lambda i, kk: (0, 0)),
                  pl.BlockSpec((cout, 1), lambda i, kk: (0, 0))],
        out_specs=[pl.BlockSpec((3, cout, tm), lambda i, kk: (0, 0, i)),
                   pl.BlockSpec((tm, 3 * cout), lambda i, kk: (i, 0))],
        compiler_params=_cp(("parallel", "arbitrary")),
    )(nbr_pm, ctr, w, s, t)


def _conv5_kernel(nbr1_ref, ctr1_ref, nbr2_ref, ctr2_ref, nbr3_ref, ctr3_ref,
                  nbr4_ref, ctr4_ref, w1_ref, wb_ref, w5_ref,
                  s1_ref, t1_ref, s2_ref, t2_ref, s3_ref, t3_ref, s4_ref, t4_ref,
                  s5_ref, t5_ref, mean_ref, *, inv_k, cout):
    kk = pl.program_id(1)
    w1 = w1_ref[...]
    wb = wb_ref[...]
    w5 = w5_ref[...]
    c = ctr2_ref.shape[1]
    st = [(s1_ref[...], t1_ref[...]), (s2_ref[...], t2_ref[...]),
          (s3_ref[...], t3_ref[...]), (s4_ref[...], t4_ref[...])]
    nb1 = _pm_rows(jnp.transpose(nbr1_ref[...]), 1)
    nb2 = _pm_rows(jnp.transpose(nbr2_ref[...]), c)
    nb3 = _pm_rows(jnp.transpose(nbr3_ref[...]), c)
    nb4 = _pm_rows(jnp.transpose(nbr4_ref[...]), c)
    pcols = [None] * 3
    dcols = [None] * 3
    for a in range(3):
        cat1 = jnp.concatenate([nb1[a], ctr1_ref[a]], axis=0)
        pd1 = jnp.dot(w1, cat1, preferred_element_type=jnp.float32)
        cat = jnp.concatenate([nb2[a], ctr2_ref[a], nb3[a], ctr3_ref[a],
                               nb4[a], ctr4_ref[a]], axis=0)
        pdb = jnp.dot(wb, cat, preferred_element_type=jnp.float32)
        pcols[a] = [pd1[:cout], pdb[:cout], pdb[2 * cout:3 * cout],
                    pdb[4 * cout:5 * cout]]
        dcols[a] = [pd1[cout:], pdb[cout:2 * cout], pdb[3 * cout:4 * cout],
                    pdb[5 * cout:]]
    o5 = []
    for i in range(4):
        s_i, t_i = st[i]
        o = _nonlin([pcols[a][i] for a in range(3)],
                    [dcols[a][i] for a in range(3)], s_i, t_i)
        o5.append(o)
    cout5 = w5.shape[0] // 2
    p5, d5 = [], []
    for a in range(3):
        cat5 = jnp.concatenate([o5[i][a] for i in range(4)], axis=0)
        pd = jnp.dot(w5, cat5, preferred_element_type=jnp.float32)
        p5.append(pd[:cout5])
        d5.append(pd[cout5:])
    o = _nonlin(p5, d5, s5_ref[...], t5_ref[...])

    @pl.when(kk == 0)
    def _():
        mean_ref[...] = jnp.zeros_like(mean_ref)

    for a in range(3):
        mean_ref[a] = mean_ref[a] + o[a]

    @pl.when(kk == pl.num_programs(1) - 1)
    def _():
        mean_ref[...] = mean_ref[...] * inv_k


def _conv5(nbrs, ctrs, w1, wb, w5, sts, s5, t5, k):
    _, c, m = ctrs[1].shape
    cout5 = w5.shape[0] // 2
    cout = wb.shape[0] // 6
    tm = _pick_tile(m)
    nt = m // tm
    nbr_spec = lambda arr: pl.BlockSpec((tm, arr.shape[1]), lambda i, kk: (kk * nt + i, 0))
    ctr_spec = lambda cc: pl.BlockSpec((3, cc, tm), lambda i, kk: (0, 0, i))
    wspec = lambda shp: pl.BlockSpec(shp, lambda i, kk: tuple(0 for _ in shp))
    in_specs = [nbr_spec(nbrs[0]), ctr_spec(1), nbr_spec(nbrs[1]), ctr_spec(c),
                nbr_spec(nbrs[2]), ctr_spec(c), nbr_spec(nbrs[3]), ctr_spec(c),
                wspec(w1.shape), wspec(wb.shape), wspec(w5.shape)]
    st_args = []
    for s_i, t_i in sts:
        in_specs += [wspec(s_i.shape), wspec(t_i.shape)]
        st_args += [s_i, t_i]
    in_specs += [wspec(s5.shape), wspec(t5.shape)]
    return pl.pallas_call(
        functools.partial(_conv5_kernel, inv_k=1.0 / k, cout=cout),
        out_shape=jax.ShapeDtypeStruct((3, cout5, m), jnp.float32),
        grid=(nt, k),
        in_specs=in_specs,
        out_specs=pl.BlockSpec((3, cout5, tm), lambda i, kk: (0, 0, i)),
        compiler_params=_cp(("parallel", "arbitrary")),
    )(nbrs[0], ctrs[0], nbrs[1], ctrs[1], nbrs[2], ctrs[2], nbrs[3], ctrs[3],
      w1, wb, w5, *st_args, s5, t5)


def _tail_kernel(x5_ref, w6_ref, s6_ref, t6_ref, wv1_ref, sv1_ref, tv1_ref,
                 wv2_ref, sv2_ref, tv2_ref, wlin_ref, w1a_ref, w1b_ref, b1_ref,
                 s1_ref, t1_ref, w2_ref, b2_ref, s2_ref, t2_ref, w3_ref, b3_ref,
                 o_ref, *, inv_n, c6):
    w6 = w6_ref[...]
    p6, d6 = [], []
    for a in range(3):
        pd = jnp.dot(w6, x5_ref[a], preferred_element_type=jnp.float32)
        p6.append(pd[:c6])
        d6.append(pd[c6:])
    x6 = _nonlin(p6, d6, s6_ref[...], t6_ref[...])
    xc = []
    for a in range(3):
        xm = jnp.sum(x6[a], axis=-1, keepdims=True) * inv_n
        xc.append(jnp.concatenate(
            [x6[a], jnp.broadcast_to(xm, x6[a].shape)], axis=0))

    wv1 = wv1_ref[...]
    cv1 = wv1.shape[0] // 2
    p, d = [], []
    for a in range(3):
        pd = jnp.dot(wv1, xc[a], preferred_element_type=jnp.float32)
        p.append(pd[:cv1])
        d.append(pd[cv1:])
    z = _nonlin(p, d, sv1_ref[...], tv1_ref[...])

    wv2 = wv2_ref[...]
    cv2 = wv2.shape[0] // 2
    p, d = [], []
    for a in range(3):
        pd = jnp.dot(wv2, z[a], preferred_element_type=jnp.float32)
        p.append(pd[:cv2])
        d.append(pd[cv2:])
    z = _nonlin(p, d, sv2_ref[...], tv2_ref[...])

    w_lin = wlin_ref[...]
    r = [jnp.dot(w_lin, z[a], preferred_element_type=jnp.float32) for a in range(3)]
    cdims = (((0,), (0,)), ((), ()))
    h = b1_ref[...]
    for kk in range(3):
        xstd = (xc[0] * r[0][kk:kk + 1, :] + xc[1] * r[1][kk:kk + 1, :]
                + xc[2] * r[2][kk:kk + 1, :])
        mx = jnp.max(xstd, axis=-1, keepdims=True)
        av = jnp.sum(xstd, axis=-1, keepdims=True) * inv_n
        h = h + lax.dot_general(mx, w1a_ref[kk], cdims,
                                preferred_element_type=jnp.float32)
        h = h + lax.dot_general(av, w1b_ref[kk], cdims,
                                preferred_element_type=jnp.float32)
    h = h * s1_ref[...] + t1_ref[...]
    h = jnp.where(h >= 0.0, h, MLP_NEG_SLOPE * h)
    h = jnp.dot(h, w2_ref[...], preferred_element_type=jnp.float32) + b2_ref[...]
    h = h * s2_ref[...] + t2_ref[...]
    h = jnp.where(h >= 0.0, h, MLP_NEG_SLOPE * h)
    o_ref[0] = jnp.dot(h, w3_ref[...], preferred_element_type=jnp.float32) + b3_ref[...]


def _tail(x5, w6, s6, t6, wv1, sv1, tv1, wv2, sv2, tv2, hp, b, n):
    _, c5, m = x5.shape
    c6 = w6.shape[0] - 1
    nc = hp['w3'].shape[1]
    ws = lambda shp: pl.BlockSpec(shp, lambda bi: tuple(0 for _ in shp))
    out = pl.pallas_call(
        functools.partial(_tail_kernel, inv_n=1.0 / n, c6=c6),
        out_shape=jax.ShapeDtypeStruct((b, 1, nc), jnp.float32),
        grid=(b,),
        in_specs=[pl.BlockSpec((3, c5, n), lambda bi: (0, 0, bi)),
                  ws(w6.shape), ws(s6.shape), ws(t6.shape),
                  ws(wv1.shape), ws(sv1.shape), ws(tv1.shape),
                  ws(wv2.shape), ws(sv2.shape), ws(tv2.shape),
                  ws(hp['std_lin'].shape), ws(hp['w1a'].shape), ws(hp['w1b'].shape),
                  ws(hp['b1'].shape), ws(hp['s1'].shape), ws(hp['t1'].shape),
                  ws(hp['w2'].shape), ws(hp['b2'].shape), ws(hp['s2'].shape),
                  ws(hp['t2'].shape), ws(hp['w3'].shape), ws(hp['b3'].shape)],
        out_specs=pl.BlockSpec((1, 1, nc), lambda bi: (bi, 0, 0)),
        compiler_params=_cp(("parallel",)),
    )(x5, w6, s6, t6, wv1, sv1, tv1, wv2, sv2, tv2,
      hp['std_lin'], hp['w1a'], hp['w1b'], hp['b1'], hp['s1'], hp['t1'],
      hp['w2'], hp['b2'], hp['s2'], hp['t2'], hp['w3'], hp['b3'])
    return out[:, 0, :]


def _stack_edge_w(wf_n, wf_c, wd_n, wd_c):
    return jnp.concatenate([jnp.concatenate([wf_n, wf_c], axis=1),
                            jnp.concatenate([wd_n, wd_c], axis=1)], axis=0)


def kernel(points,
           conv1_wf_n, conv1_wf_c, conv1_wd_n, conv1_wd_c, conv1_s, conv1_t,
           conv2_wf_n, conv2_wf_c, conv2_wd_n, conv2_wd_c, conv2_s, conv2_t,
           conv3_wf_n, conv3_wf_c, conv3_wd_n, conv3_wd_c, conv3_s, conv3_t,
           conv4_wf_n, conv4_wf_c, conv4_wd_n, conv4_wd_c, conv4_s, conv4_t,
           conv5_wf, conv5_wd, conv5_s, conv5_t,
           conv6_wf, conv6_wd, conv6_s, conv6_t,
           stdvn1_wf, stdvn1_wd, stdvn1_s, stdvn1_t,
           stdvn2_wf, stdvn2_wd, stdvn2_s, stdvn2_t,
           head_std_lin, head_w1a, head_w1b, head_b1, head_s1, head_t1,
           head_w2, head_b2, head_s2, head_t2, head_w3, head_b3):
    k = 8
    b, three, n = points.shape
    m = b * n
    x = jnp.transpose(points, (1, 0, 2)).reshape(3, 1, m)
    x_pm = jnp.transpose(points, (0, 2, 1)).reshape(m, 3)

    w1 = _stack_edge_w(conv1_wf_n, conv1_wf_c, conv1_wd_n, conv1_wd_c)
    w2 = _stack_edge_w(conv2_wf_n, conv2_wf_c, conv2_wd_n, conv2_wd_c)
    w3 = _stack_edge_w(conv3_wf_n, conv3_wf_c, conv3_wd_n, conv3_wd_c)
    w4 = _stack_edge_w(conv4_wf_n, conv4_wf_c, conv4_wd_n, conv4_wd_c)
    cout = conv2_wf_n.shape[0]
    cc = conv2_wf_n.shape[1]
    wb = jnp.zeros((6 * cout, 6 * cc), jnp.float32)
    wb = wb.at[0:2 * cout, 0:2 * cc].set(w2)
    wb = wb.at[2 * cout:4 * cout, 2 * cc:4 * cc].set(w3)
    wb = wb.at[4 * cout:6 * cout, 4 * cc:6 * cc].set(w4)
    cout5 = conv5_wf.shape[1]
    w5 = jnp.concatenate(
        [jnp.transpose(conv5_wf, (1, 0, 2)).reshape(cout5, -1),
         jnp.transpose(conv5_wd, (1, 0, 2)).reshape(cout5, -1)], axis=0)

    nbr1 = _knn_gather(x, x_pm, k, b, n)
    x1, x1_pm = _edge_mean(nbr1, x, w1, conv1_s, conv1_t, k)
    nbr2 = _knn_gather(x1, x1_pm, k, b, n)
    x2, x2_pm = _edge_mean(nbr2, x1, w2, conv2_s, conv2_t, k)
    nbr3 = _knn_gather(x2, x2_pm, k, b, n)
    x3, x3_pm = _edge_mean(nbr3, x2, w3, conv3_s, conv3_t, k)
    nbr4 = _knn_gather(x3, x3_pm, k, b, n)

    x5 = _conv5([nbr1, nbr2, nbr3, nbr4], [x, x1, x2, x3], w1, wb, w5,
                [(conv1_s, conv1_t), (conv2_s, conv2_t),
                 (conv3_s, conv3_t), (conv4_s, conv4_t)],
                conv5_s, conv5_t, k)

    w6 = jnp.concatenate([conv6_wf, conv6_wd], axis=0)
    wv1 = jnp.concatenate([stdvn1_wf, stdvn1_wd], axis=0)
    wv2 = jnp.concatenate([stdvn2_wf, stdvn2_wd], axis=0)
    hp = dict(std_lin=head_std_lin, w1a=head_w1a, w1b=head_w1b, b1=head_b1,
              s1=head_s1, t1=head_t1, w2=head_w2, b2=head_b2, s2=head_s2,
              t2=head_t2, w3=head_w3, b3=head_b3)
    return _tail(x5, w6, conv6_s, conv6_t, wv1, stdvn1_s, stdvn1_t,
                 wv2, stdvn2_s, stdvn2_t, hp, b, n)
